# baseline (speedup 1.0000x reference)
import jax
import jax.numpy as jnp
from jax.experimental import pallas as pl
from jax.experimental.pallas import tpu as pltpu

_TRADE_LAMBDA = 0.25
_B = 64
_N = 1024
_K = 204
_KPAD = 208
_MINT = -(1 << 31)
_BPS = 64


def _cumsum_lanes(x):
    r, c = x.shape
    sh = 1
    while sh < c:
        z = jnp.zeros((r, sh), x.dtype)
        x = x + jnp.concatenate([z, x[:, :c - sh]], axis=1)
        sh *= 2
    return x


def _select(y):
    y = jnp.where(y == 0.0, 0.0, y)
    bits = jax.lax.bitcast_convert_type(y, jnp.int32)
    key = jnp.where(bits >= 0, bits, ~(bits & 0x7FFFFFFF))
    k2 = jnp.concatenate([key, ~key], axis=0)

    tu = jnp.zeros((2 * _B, 1), jnp.int32)
    for bit in range(31, -1, -1):
        bv = jnp.asarray(_MINT if bit == 31 else (1 << bit), jnp.int32)
        cand = tu | bv
        scand = cand ^ _MINT
        cnt = jnp.sum((k2 >= scand).astype(jnp.int32), axis=1, keepdims=True)
        tu = jnp.where(cnt >= _K, cand, tu)
    thr = tu ^ _MINT

    gti = (k2 > thr).astype(jnp.int32)
    eqi = (k2 == thr).astype(jnp.int32)
    g = jnp.sum(gti, axis=1, keepdims=True)
    need = _K - g

    incl = _cumsum_lanes(eqi)
    tot = jnp.sum(eqi, axis=1, keepdims=True)
    pos_l = incl - eqi
    pos_r = tot - incl
    rowi = jax.lax.broadcasted_iota(jnp.int32, (2 * _B, _N), 0)
    tie_pos = jnp.where(rowi < _B, pos_l, pos_r)
    sel = gti + eqi * (tie_pos < need).astype(jnp.int32)

    slot = _cumsum_lanes(sel) - sel
    return jnp.where(sel > 0, slot, -1)


def _fused_kernel(y_ref, s_ref, p_ref, yt_ref, w_ref, m_ref,
                  out_ref, slots_ref, acc_ref):
    i = pl.program_id(0)

    @pl.when(i == 0)
    def _select_step():
        for j in range(6):
            acc_ref[j] = 0.0
        slots_ref[...] = _select(y_ref[...])

    @pl.when(i > 0)
    def _loss_step():
        base = pl.multiple_of((i - 1) * _BPS, _BPS)
        pt_blk = slots_ref[pl.ds(base, _BPS), :]
        pb_blk = slots_ref[pl.ds(_B + base, _BPS), :]

        r256 = jax.lax.broadcasted_iota(jnp.int32, (_KPAD, _N), 0)
        dnum_c = (((1,), (1,)), ((), ()))
        rank_part = 0.0
        for t in range(_BPS):
            sw = jnp.concatenate([s_ref[t:t + 1, :], w_ref[t:t + 1, :]],
                                 axis=0)
            p_top = (pt_blk[t:t + 1, :] == r256).astype(jnp.bfloat16)
            p_bot = (pb_blk[t:t + 1, :] == r256).astype(jnp.bfloat16)
            top_g = jax.lax.dot_general(p_top, sw, dnum_c,
                                        preferred_element_type=jnp.float32)
            bot_g = jax.lax.dot_general(sw, p_bot, dnum_c,
                                        preferred_element_type=jnp.float32)
            st = top_g[:, 0:1]
            at = jnp.sqrt(top_g[:, 1:2])
            sb = bot_g[0:1, :]
            ab = jnp.sqrt(bot_g[1:2, :])

            d = jnp.minimum(sb - st, 60.0)
            sp = jnp.log1p(jnp.exp(d))
            wp = at * ab
            num = jnp.sum(sp * wp)
            den = jnp.sum(at) * jnp.sum(ab)
            rank_part += num / (den + 1e-8)

        w = w_ref[...]
        p = p_ref[...]
        ytr = yt_ref[...]
        m = m_ref[...]
        logp = jnp.maximum(jnp.log(p), -100.0)
        log1mp = jnp.maximum(jnp.log(1.0 - p), -100.0)
        bce = -(ytr * logp + (1.0 - ytr) * log1mp)
        mw = w * m
        t_den = jnp.sum(mw, axis=1, keepdims=True)
        t_num = jnp.sum(bce * mw, axis=1, keepdims=True)
        validf = (t_den > 0.0).astype(jnp.float32)
        pb_trade = t_num / (t_den + 1e-8)

        acc_ref[0] += rank_part
        acc_ref[1] += jnp.sum(validf * pb_trade)
        acc_ref[2] += jnp.sum(validf)
        acc_ref[3] += jnp.sum(p * m)
        acc_ref[4] += jnp.sum(m)

    @pl.when(i == _B // _BPS)
    def _finish():
        avg_rank = acc_ref[0] / float(_B)
        avg_trade = acc_ref[1] / jnp.maximum(acc_ref[2], 1.0)
        out_ref[0] = avg_rank + _TRADE_LAMBDA * avg_trade
        out_ref[1] = avg_rank
        out_ref[2] = avg_trade
        out_ref[3] = acc_ref[3] / jnp.maximum(acc_ref[4], 1.0)


def kernel(scores, p_trade, y_rank, y_trade, weights, mask):
    yspec = pl.BlockSpec((_B, _N), lambda i: (0, 0))
    blk = pl.BlockSpec((_BPS, _N), lambda i: (jnp.maximum(i - 1, 0), 0))
    args = (y_rank, scores, p_trade, y_trade, weights,
            mask.astype(jnp.float32))
    out = pl.pallas_call(
        _fused_kernel,
        grid=(1 + _B // _BPS,),
        in_specs=[yspec] + [blk] * 5,
        out_specs=pl.BlockSpec(memory_space=pltpu.SMEM),
        out_shape=jax.ShapeDtypeStruct((4,), jnp.float32),
        scratch_shapes=[pltpu.VMEM((2 * _B, _N), jnp.int32),
                        pltpu.SMEM((6,), jnp.float32)],
    )(*args)
    return (out[0], out[1], out[2], out[3])


# speedup vs baseline: 1.0738x; 1.0738x over previous
import jax
import jax.numpy as jnp
from jax.experimental import pallas as pl
from jax.experimental.pallas import tpu as pltpu

_TRADE_LAMBDA = 0.25
_B = 64
_N = 1024
_K = 204
_KPAD = 208
_MINT = -(1 << 31)
_BPS = 64


def _cumsum_lanes(x):
    r, c = x.shape
    sh = 1
    while sh < c:
        z = jnp.zeros((r, sh), x.dtype)
        x = x + jnp.concatenate([z, x[:, :c - sh]], axis=1)
        sh *= 2
    return x


def _select(y):
    y = jnp.where(y == 0.0, 0.0, y)
    bits = jax.lax.bitcast_convert_type(y, jnp.int32)
    key = jnp.where(bits >= 0, bits, ~(bits & 0x7FFFFFFF))
    k2 = jnp.concatenate([key, ~key], axis=0)

    tu = jnp.zeros((2 * _B, 1), jnp.int32)
    for bit in range(31, -1, -1):
        bv = jnp.asarray(_MINT if bit == 31 else (1 << bit), jnp.int32)
        cand = tu | bv
        scand = cand ^ _MINT
        cnt = jnp.sum((k2 >= scand).astype(jnp.int32), axis=1, keepdims=True)
        tu = jnp.where(cnt >= _K, cand, tu)
    thr = tu ^ _MINT

    gti = (k2 > thr).astype(jnp.int32)
    eqi = (k2 == thr).astype(jnp.int32)
    g = jnp.sum(gti, axis=1, keepdims=True)
    need = _K - g

    incl = _cumsum_lanes(eqi)
    tot = jnp.sum(eqi, axis=1, keepdims=True)
    pos_l = incl - eqi
    pos_r = tot - incl
    rowi = jax.lax.broadcasted_iota(jnp.int32, (2 * _B, _N), 0)
    tie_pos = jnp.where(rowi < _B, pos_l, pos_r)
    sel = gti + eqi * (tie_pos < need).astype(jnp.int32)

    slot = _cumsum_lanes(sel) - sel
    return jnp.where(sel > 0, slot, -1)


def _fused_kernel(y_ref, s_ref, p_ref, yt_ref, w_ref, m_ref,
                  out_ref, slots_ref, acc_ref):
    i = pl.program_id(0)

    @pl.when(i == 0)
    def _select_step():
        for j in range(6):
            acc_ref[j] = 0.0
        slots_ref[...] = _select(y_ref[...])

    @pl.when(i > 0)
    def _loss_step():
        base = pl.multiple_of((i - 1) * _BPS, _BPS)
        pt_blk = slots_ref[pl.ds(base, _BPS), :]
        pb_blk = slots_ref[pl.ds(_B + base, _BPS), :]

        r256 = jax.lax.broadcasted_iota(jnp.int32, (_KPAD, _N), 0)
        dnum_c = (((1,), (1,)), ((), ()))
        log2e = 1.4426950408889634
        ln2 = 0.6931471805599453
        wsq = jnp.sqrt(w_ref[...])
        s2 = s_ref[...] * log2e
        rank_part = 0.0
        for t in range(_BPS):
            sw = jnp.concatenate([s2[t:t + 1, :], wsq[t:t + 1, :]],
                                 axis=0)
            p_top = (pt_blk[t:t + 1, :] == r256).astype(jnp.float32)
            p_bot = (pb_blk[t:t + 1, :] == r256).astype(jnp.float32)
            top_g = jax.lax.dot_general(p_top, sw, dnum_c,
                                        preferred_element_type=jnp.float32)
            bot_g = jax.lax.dot_general(sw, p_bot, dnum_c,
                                        preferred_element_type=jnp.float32)
            st2 = top_g[:, 0:1]
            at = top_g[:, 1:2]
            sb2 = bot_g[0:1, :]
            ab = bot_g[1:2, :]

            d2 = jnp.minimum(sb2 - st2, 86.0)
            sp2 = jnp.log2(1.0 + jnp.exp2(d2))
            wp = at * ab
            num = jnp.sum(sp2 * wp) * ln2
            den = jnp.sum(at) * jnp.sum(ab)
            rank_part += num / (den + 1e-8)

        w = w_ref[...]
        p = p_ref[...]
        ytr = yt_ref[...]
        m = m_ref[...]
        logp = jnp.maximum(jnp.log(p), -100.0)
        log1mp = jnp.maximum(jnp.log(1.0 - p), -100.0)
        bce = -(ytr * logp + (1.0 - ytr) * log1mp)
        mw = w * m
        t_den = jnp.sum(mw, axis=1, keepdims=True)
        t_num = jnp.sum(bce * mw, axis=1, keepdims=True)
        validf = (t_den > 0.0).astype(jnp.float32)
        pb_trade = t_num / (t_den + 1e-8)

        acc_ref[0] += rank_part
        acc_ref[1] += jnp.sum(validf * pb_trade)
        acc_ref[2] += jnp.sum(validf)
        acc_ref[3] += jnp.sum(p * m)
        acc_ref[4] += jnp.sum(m)

    @pl.when(i == _B // _BPS)
    def _finish():
        avg_rank = acc_ref[0] / float(_B)
        avg_trade = acc_ref[1] / jnp.maximum(acc_ref[2], 1.0)
        out_ref[0] = avg_rank + _TRADE_LAMBDA * avg_trade
        out_ref[1] = avg_rank
        out_ref[2] = avg_trade
        out_ref[3] = acc_ref[3] / jnp.maximum(acc_ref[4], 1.0)


def kernel(scores, p_trade, y_rank, y_trade, weights, mask):
    yspec = pl.BlockSpec((_B, _N), lambda i: (0, 0))
    blk = pl.BlockSpec((_BPS, _N), lambda i: (jnp.maximum(i - 1, 0), 0))
    args = (y_rank, scores, p_trade, y_trade, weights,
            mask.astype(jnp.float32))
    out = pl.pallas_call(
        _fused_kernel,
        grid=(1 + _B // _BPS,),
        in_specs=[yspec] + [blk] * 5,
        out_specs=pl.BlockSpec(memory_space=pltpu.SMEM),
        out_shape=jax.ShapeDtypeStruct((4,), jnp.float32),
        scratch_shapes=[pltpu.VMEM((2 * _B, _N), jnp.int32),
                        pltpu.SMEM((6,), jnp.float32)],
    )(*args)
    return (out[0], out[1], out[2], out[3])
